# Initial kernel scaffold; baseline (speedup 1.0000x reference)
import jax
import jax.numpy as jnp
from jax.experimental import pallas as pl


def kernel(X, X_spa, dynamic_graph, static_graph, W_fc, W_gcn, b_gcn, W_bc, b_bc, W_fk, b_fk, ln_scale, ln_bias):
    raise NotImplementedError("write your pallas kernel here")


# speedup vs baseline: 2.2772x; 2.2772x over previous
import jax
import jax.numpy as jnp
from jax.experimental import pallas as pl
from jax.experimental.pallas import tpu as pltpu

K_T = 3
K_S = 2
HIDDEN = 64
FK_DIM = 256
SEQ_LENGTH = 12
GAP = 2

_F32 = jnp.float32


def _dot(a, b):
    return jnp.dot(a, b, preferred_element_type=_F32)


def _dif_block_kernel(xc_ref, xspa_ref, dyn_ref, wfcT_ref, wgT_ref, bg_ref,
                      wbcT_ref, bbc_ref, wfkT_ref, bfk_ref, lns_ref, lnb_ref,
                      u_ref, fh_ref, xk_scr, x0_scr):
    L = xspa_ref.shape[1]
    N = xspa_ref.shape[2]
    D = HIDDEN
    Lp = L - K_T + 1
    STEPS = SEQ_LENGTH // GAP - 1

    WfcT = wfcT_ref[...]
    Wg = wgT_ref[...]
    bg = bg_ref[...]
    WbcT = wbcT_ref[...]
    bbc = bbc_ref[...]
    WfkT = wfkT_ref[...]
    bfk = bfk_ref[...]
    lns = lns_ref[...]
    lnb = lnb_ref[...]

    dynC = dyn_ref[...].reshape(K_S * N, K_T * N)

    for l in range(Lp):
        xr = jnp.concatenate(
            [xspa_ref[0, l + j] for j in range(K_T)], axis=-1)
        t = jnp.tanh(_dot(xr, WfcT))
        x0_scr[l] = (t[:, 0:D] + t[:, D:2 * D] + t[:, 2 * D:3 * D]) * (1.0 / K_T)
        for j in range(K_T):
            xk_scr[j * N:(j + 1) * N, l * D:(l + 1) * D] = t[:, j * D:(j + 1) * D]

    acat = _dot(dynC, xk_scr[...])

    zs_last = None
    for l in range(Lp):
        h = jnp.concatenate(
            [x0_scr[l], acat[0:N, l * D:(l + 1) * D],
             acat[N:2 * N, l * D:(l + 1) * D]], axis=-1)
        z = _dot(h, Wg) + bg
        if l == Lp - 1:
            zs_last = z
        bc = _dot(z, WbcT) + bbc
        v = xc_ref[0, l] - jnp.maximum(bc, 0.0)
        mu = jnp.mean(v, axis=-1, keepdims=True)
        var = jnp.mean((v - mu) * (v - mu), axis=-1, keepdims=True)
        u_ref[0, l] = (v - mu) * jax.lax.rsqrt(var + 1e-5) * lns + lnb

    def conv_one(w0, w1, w2):
        xr = jnp.concatenate([w0, w1, w2], axis=-1)
        t = jnp.tanh(_dot(xr, WfcT))
        x0 = (t[:, 0:D] + t[:, D:2 * D] + t[:, 2 * D:3 * D]) * (1.0 / K_T)
        xk = jnp.concatenate(
            [t[:, j * D:(j + 1) * D] for j in range(K_T)], axis=0)
        ac = _dot(dynC, xk)
        h = jnp.concatenate([x0, ac[0:N], ac[N:2 * N]], axis=-1)
        return _dot(h, Wg) + bg

    wins = [xspa_ref[0, L - 2], xspa_ref[0, L - 1], zs_last]
    frames = [zs_last]
    for _ in range(STEPS):
        f = conv_one(wins[-3], wins[-2], wins[-1])
        wins.append(f)
        frames.append(f)

    fcat = jnp.concatenate(frames, axis=0)
    fh = _dot(fcat, WfkT) + bfk
    fh_ref[0] = fh.reshape(STEPS + 1, N, FK_DIM)


def kernel(X, X_spa, dynamic_graph, static_graph, W_fc, W_gcn, b_gcn, W_bc,
           b_bc, W_fk, b_fk, ln_scale, ln_bias):
    B, L, N, D = X_spa.shape
    Lp = L - K_T + 1
    S = SEQ_LENGTH // GAP

    Xc = X[:, -Lp:]
    args = (
        Xc,
        X_spa,
        dynamic_graph,
        W_fc.T,
        W_gcn.T,
        b_gcn.reshape(1, D),
        W_bc.T,
        b_bc.reshape(1, D),
        W_fk.T,
        b_fk.reshape(1, FK_DIM),
        ln_scale.reshape(1, D),
        ln_bias.reshape(1, D),
    )

    full = lambda shape: pl.BlockSpec(shape, lambda b: (0,) * len(shape))
    batched = lambda shape: pl.BlockSpec(
        (1,) + shape[1:], lambda b: (b,) + (0,) * (len(shape) - 1))

    in_specs = [
        batched(Xc.shape),
        batched(X_spa.shape),
        pl.BlockSpec((K_S, 1, N, K_T * N), lambda b: (0, b, 0, 0)),
        full((K_T * D, K_T * D)),
        full((K_T * D, D)),
        full((1, D)),
        full((D, D)),
        full((1, D)),
        full((D, FK_DIM)),
        full((1, FK_DIM)),
        full((1, D)),
        full((1, D)),
    ]
    out_specs = [
        batched((B, Lp, N, D)),
        batched((B, S, N, FK_DIM)),
    ]
    out_shape = [
        jax.ShapeDtypeStruct((B, Lp, N, D), _F32),
        jax.ShapeDtypeStruct((B, S, N, FK_DIM), _F32),
    ]

    u, fh = pl.pallas_call(
        _dif_block_kernel,
        grid=(B,),
        in_specs=in_specs,
        out_specs=out_specs,
        out_shape=out_shape,
        scratch_shapes=[
            pltpu.VMEM((K_T * N, Lp * D), _F32),
            pltpu.VMEM((Lp, N, D), _F32),
        ],
    )(*args)
    return (u, fh)
